# baseline (speedup 1.0000x reference)
import functools

import jax
import jax.numpy as jnp
from jax import lax
from jax.experimental import pallas as pl
from jax.experimental.pallas import tpu as pltpu
from jax.experimental.pallas import tpu_sc as plsc

N_NODES = 10000
D_FEAT = 128
N_EDGES = 320000

NC = 2
NS = 16

EDGES_PER_TILE = N_EDGES // NS
CHUNK = 40
NCHUNK = EDGES_PER_TILE // CHUNK
IDXB = 50
NBLK = NCHUNK // IDXB
NBUF = 7
ROWS_MAIN = 624
TAIL0 = NS * ROWS_MAIN
TAIL = N_NODES - TAIL0

_mesh = plsc.VectorSubcoreMesh(core_axis_name="c", subcore_axis_name="s")


@functools.partial(
    pl.kernel,
    mesh=_mesh,
    out_type=jax.ShapeDtypeStruct((NC, N_NODES, D_FEAT), jnp.float32),
    scratch_types=[
        pltpu.VMEM((IDXB, CHUNK), jnp.int32),
        pltpu.VMEM((IDXB, CHUNK), jnp.int32),
        pltpu.VMEM((NBUF, CHUNK, D_FEAT), jnp.float32),
        pltpu.VMEM_SHARED((N_NODES, D_FEAT), jnp.float32),
        pltpu.SemaphoreType.DMA((NBUF,)),
        pltpu.SemaphoreType.DMA((NBUF,)),
    ],
)
def _spmm2(x_hbm, srcs_hbm, dsts_hbm, zeros_hbm, out_hbm,
           src_v, dst_v, rows_v, acc_sh, gsem, ssem):
    c = lax.axis_index("c")
    s = lax.axis_index("s")

    row0 = s * ROWS_MAIN
    pltpu.sync_copy(zeros_hbm.at[pl.ds(row0, ROWS_MAIN)],
                    acc_sh.at[pl.ds(row0, ROWS_MAIN)])

    @pl.when(s == NS - 1)
    def _zero_tail():
        pltpu.sync_copy(zeros_hbm.at[pl.ds(TAIL0, TAIL)],
                        acc_sh.at[pl.ds(TAIL0, TAIL)])

    plsc.subcore_barrier()

    def gather(j, k):
        pltpu.async_copy(x_hbm.at[src_v.at[j]], rows_v.at[k], gsem.at[k])

    def wait_gather(j, k):
        pltpu.make_async_copy(x_hbm.at[src_v.at[j]], rows_v.at[k],
                              gsem.at[k]).wait()

    def scatter(j, k):
        pltpu.async_copy(rows_v.at[k], acc_sh.at[dst_v.at[j]], ssem.at[k],
                         add=True)

    def wait_scatter(j, k):
        pltpu.make_async_copy(rows_v.at[k], acc_sh.at[dst_v.at[j]],
                              ssem.at[k]).wait()

    def block(b, carry):
        pltpu.sync_copy(srcs_hbm.at[c, s, b], src_v)
        pltpu.sync_copy(dsts_hbm.at[c, s, b], dst_v)

        for j in range(4):
            gather(j, j)
        for j in range(IDXB):
            k = j % NBUF
            wait_gather(j, k)
            scatter(j, k)
            nj = j + 4
            if nj < IDXB:
                kn = nj % NBUF
                if nj >= NBUF:
                    wait_scatter(nj - NBUF, kn)
                gather(nj, kn)
        for j in range(max(0, IDXB - NBUF), IDXB):
            wait_scatter(j, j % NBUF)
        return carry

    lax.fori_loop(0, NBLK, block, 0)

    plsc.subcore_barrier()
    pltpu.sync_copy(acc_sh.at[pl.ds(row0, ROWS_MAIN)],
                    out_hbm.at[c, pl.ds(row0, ROWS_MAIN)])

    @pl.when(s == NS - 1)
    def _out_tail():
        pltpu.sync_copy(acc_sh.at[pl.ds(TAIL0, TAIL)],
                        out_hbm.at[c, pl.ds(TAIL0, TAIL)])


def kernel(x, adj_t, adj_t2):
    srcs = jnp.stack([adj_t[1], adj_t2[1]]).reshape(NC, NS, NBLK, IDXB, CHUNK)
    dsts = jnp.stack([adj_t[0], adj_t2[0]]).reshape(NC, NS, NBLK, IDXB, CHUNK)
    zeros = jnp.zeros((N_NODES, D_FEAT), jnp.float32)
    out = _spmm2(x, srcs, dsts, zeros)
    return jnp.concatenate([out[0], out[1]], axis=1)


# speedup vs baseline: 1.1814x; 1.1814x over previous
import functools

import jax
import jax.numpy as jnp
from jax import lax
from jax.experimental import pallas as pl
from jax.experimental.pallas import tpu as pltpu
from jax.experimental.pallas import tpu_sc as plsc

N_NODES = 10000
D_FEAT = 128
N_EDGES = 320000

NC = 2
NS = 16

EDGES_PER_TILE = N_EDGES // NS
CHUNK = 80
NCHUNK = EDGES_PER_TILE // CHUNK
IDXB = 25
NBLK = NCHUNK // IDXB
NBUF = 4
ROWS_MAIN = 624
TAIL0 = NS * ROWS_MAIN
TAIL = N_NODES - TAIL0
ZCHUNKS = (80, 80, 80, 80, 80, 80, 80, 64)

_mesh = plsc.VectorSubcoreMesh(core_axis_name="c", subcore_axis_name="s")


@functools.partial(
    pl.kernel,
    mesh=_mesh,
    out_type=jax.ShapeDtypeStruct((N_NODES, NC * D_FEAT), jnp.float32),
    scratch_types=[
        pltpu.VMEM((IDXB, CHUNK), jnp.int32),
        pltpu.VMEM((IDXB, CHUNK), jnp.int32),
        pltpu.VMEM((NBUF, CHUNK, D_FEAT), jnp.float32),
        pltpu.VMEM_SHARED((N_NODES, D_FEAT), jnp.float32),
        pltpu.SemaphoreType.DMA((NBUF,)),
        pltpu.SemaphoreType.DMA((NBUF,)),
    ],
)
def _spmm2(x_hbm, adj1_hbm, adj2_hbm, out_hbm,
           src_v, dst_v, rows_v, acc_sh, gsem, ssem):
    c = lax.axis_index("c")
    s = lax.axis_index("s")

    row0 = s * ROWS_MAIN

    zvec = jnp.zeros((16,), jnp.float32)

    def zrow(r, carry):
        for i in range(D_FEAT // 16):
            rows_v[0, r, pl.ds(i * 16, 16)] = zvec
        return carry

    lax.fori_loop(0, CHUNK, zrow, 0)
    off = 0
    for zc in ZCHUNKS:
        pltpu.sync_copy(rows_v.at[0, pl.ds(0, zc)],
                        acc_sh.at[pl.ds(row0 + off, zc)])
        off += zc

    @pl.when(s == NS - 1)
    def _zero_tail():
        pltpu.sync_copy(rows_v.at[0, pl.ds(0, TAIL)],
                        acc_sh.at[pl.ds(TAIL0, TAIL)])

    plsc.subcore_barrier()

    def gather(j, k):
        pltpu.async_copy(x_hbm.at[src_v.at[j]], rows_v.at[k], gsem.at[k])

    def wait_gather(j, k):
        pltpu.make_async_copy(x_hbm.at[src_v.at[j]], rows_v.at[k],
                              gsem.at[k]).wait()

    def scatter(j, k):
        pltpu.async_copy(rows_v.at[k], acc_sh.at[dst_v.at[j]], ssem.at[k],
                         add=True)

    def wait_scatter(j, k):
        pltpu.make_async_copy(rows_v.at[k], acc_sh.at[dst_v.at[j]],
                              ssem.at[k]).wait()

    def block(b, carry):
        @pl.when(c == 0)
        def _stage1():
            pltpu.sync_copy(adj1_hbm.at[1, s, b], src_v)
            pltpu.sync_copy(adj1_hbm.at[0, s, b], dst_v)

        @pl.when(c == 1)
        def _stage2():
            pltpu.sync_copy(adj2_hbm.at[1, s, b], src_v)
            pltpu.sync_copy(adj2_hbm.at[0, s, b], dst_v)

        gather(0, 0)
        gather(1, 1)
        gather(2, 2)
        for j in range(IDXB):
            k = j % NBUF
            wait_gather(j, k)
            scatter(j, k)
            nj = j + 3
            if nj < IDXB:
                kn = nj % NBUF
                if nj >= NBUF:
                    wait_scatter(nj - NBUF, kn)
                gather(nj, kn)
        for j in range(IDXB - NBUF, IDXB):
            wait_scatter(j, j % NBUF)
        return carry

    lax.fori_loop(0, NBLK, block, 0)

    plsc.subcore_barrier()
    col0 = pl.multiple_of(c * D_FEAT, D_FEAT)
    pltpu.sync_copy(acc_sh.at[pl.ds(row0, ROWS_MAIN)],
                    out_hbm.at[pl.ds(row0, ROWS_MAIN), pl.ds(col0, D_FEAT)])

    @pl.when(s == NS - 1)
    def _out_tail():
        pltpu.sync_copy(acc_sh.at[pl.ds(TAIL0, TAIL)],
                        out_hbm.at[pl.ds(TAIL0, TAIL), pl.ds(col0, D_FEAT)])


def kernel(x, adj_t, adj_t2):
    adj1 = adj_t.reshape(2, NS, NBLK, IDXB, CHUNK)
    adj2 = adj_t2.reshape(2, NS, NBLK, IDXB, CHUNK)
    return _spmm2(x, adj1, adj2)
